# baseline (speedup 1.0000x reference)
import functools

import jax
import jax.numpy as jnp
from jax import lax
from jax.experimental import pallas as pl
from jax.experimental.pallas import tpu as pltpu
from jax.experimental.pallas import tpu_sc as plsc

N = 8192
IN_F = 512
COMP = 128
K = 16
PPAD = 16
RB = 256
PB = 256
NK = N * K


CW = 256
NCH = N // CW
CPAD = 128


def _encode_knn_body(cs_ref, ce_ref, multi_ref, cxlo_ref, cxhi_ref,
                     f_ref, posb_ref, posT_ref, bcol_ref, brow_ref,
                     we_ref, be_ref, wq_ref, bq_ref, wk_ref, bk_ref,
                     wv_ref, bv_ref, wp1_ref,
                     q_ref, kvp_ref, idx_ref):
    h = jnp.dot(f_ref[...], we_ref[...], preferred_element_type=jnp.float32)
    h = h + be_ref[...]
    q_ref[...] = jnp.dot(h, wq_ref[...], preferred_element_type=jnp.float32) + bq_ref[...]
    kvp_ref[:, 0:COMP] = jnp.dot(h, wk_ref[...], preferred_element_type=jnp.float32) + bk_ref[...]
    kvp_ref[:, COMP:2 * COMP] = jnp.dot(h, wv_ref[...], preferred_element_type=jnp.float32) + bv_ref[...]

    posb = posb_ref[...]
    kvp_ref[:, 2 * COMP:3 * COMP] = jnp.dot(posb, wp1_ref[...], preferred_element_type=jnp.float32)
    sqb = jnp.sum(posb * posb, axis=1, keepdims=True)
    bcol = bcol_ref[...]
    i = pl.program_id(0)
    cs = cs_ref[i]
    ce = ce_ref[i]
    multi = multi_ref[i]
    xcol = posb[:, 0:1]
    x_lo = jnp.min(xcol)
    x_hi = jnp.max(xcol)

    init_v = jnp.full((RB, K), jnp.inf, jnp.float32)
    init_i = jnp.broadcast_to(
        (-1.0 - lax.broadcasted_iota(jnp.int32, (1, K), 1).astype(jnp.float32)),
        (RB, K))
    iota_c = lax.broadcasted_iota(jnp.int32, (RB, CW), 1)

    def merge_chunk(c, carry):
        vals, idxs = carry
        pcols = posT_ref[:, pl.ds(c * CW, CW)]
        dots = jnp.dot(posb, pcols, preferred_element_type=jnp.float32)
        sqr = jnp.sum(pcols * pcols, axis=0, keepdims=True)
        d2 = sqb + sqr - 2.0 * dots
        same = bcol == brow_ref[:, pl.ds(c * CW, CW)]
        cand = jnp.where(same, d2, jnp.inf)
        cidx = (c * CW + iota_c).astype(jnp.float32)
        wv = jnp.concatenate([vals, cand], axis=1)
        wi = jnp.concatenate([idxs, cidx], axis=1)
        nv, ni = [], []
        for _ in range(K):
            m = jnp.min(wv, axis=1, keepdims=True)
            tie = jnp.where(wv == m, wi, jnp.float32(N))
            amin = jnp.min(tie, axis=1, keepdims=True)
            nv.append(m)
            ni.append(amin)
            wv = jnp.where(tie == amin, jnp.inf, wv)
        return jnp.concatenate(nv, axis=1), jnp.concatenate(ni, axis=1)

    def scan_body(st):
        k, nb, limit, vals, idxs = st
        c = jnp.where(k == 0, i,
                      jnp.where(k - 1 < nb, i - k, i + (k - nb)))
        vals, idxs = merge_chunk(c, (vals, idxs))
        t = jnp.max(vals[:, K - 1])

        def fwd_scan(cc, e):
            gap = jnp.maximum(cxlo_ref[cc] - x_hi, 0.0)
            return jnp.minimum(e, jnp.where(gap * gap > t, cc, ce))

        def bwd_scan(cc, b):
            gap = jnp.maximum(x_lo - cxhi_ref[cc], 0.0)
            return jnp.maximum(b, jnp.where(gap * gap > t, cc + 1, cs))

        fwd_end = lax.fori_loop(i + 1, ce, fwd_scan, ce)
        bwd_begin = lax.fori_loop(cs, i, bwd_scan, cs)
        fwd_end = jnp.where(multi != 0, ce, fwd_end)
        bwd_begin = jnp.where(multi != 0, cs, bwd_begin)
        first = k == 0
        nb = jnp.where(first, i - bwd_begin, nb)
        limit = jnp.where(first, 1 + nb + fwd_end - (i + 1), limit)
        return k + 1, nb, limit, vals, idxs

    def scan_cond(st):
        k, _, limit, _, _ = st
        return k < limit

    _, _, _, _, idxs = lax.while_loop(
        scan_cond, scan_body, (0, 0, 1, init_v, init_i))
    idx_ref[...] = jnp.clip(idxs, 0.0, jnp.float32(N - 1)).astype(jnp.int32)


def _encode_knn(cs, ce, multi, cxlo, cxhi, features, pos16, posT, bcol, brow,
                W_enc, b_enc, Wq, bq, Wk, bk, Wv, bv, Wp1p):
    grid = (N // RB,)
    full = lambda shape: pl.BlockSpec(shape, lambda i, *_: (0, 0))
    rowblk = lambda w: pl.BlockSpec((RB, w), lambda i, *_: (i, 0))
    return pl.pallas_call(
        _encode_knn_body,
        grid_spec=pltpu.PrefetchScalarGridSpec(
            num_scalar_prefetch=5,
            grid=grid,
            in_specs=[
                rowblk(IN_F),
                rowblk(PPAD),
                full((PPAD, N)),
                pl.BlockSpec((RB, 1), lambda i, *_: (i, 0)),
                full((1, N)),
                full((IN_F, COMP)), full((1, COMP)),
                full((COMP, COMP)), full((1, COMP)),
                full((COMP, COMP)), full((1, COMP)),
                full((COMP, COMP)), full((1, COMP)),
                full((PPAD, COMP)),
            ],
            out_specs=[
                rowblk(COMP), rowblk(3 * COMP),
                pl.BlockSpec((RB, K), lambda i, *_: (i, 0)),
            ],
        ),
        out_shape=[
            jax.ShapeDtypeStruct((N, COMP), jnp.float32),
            jax.ShapeDtypeStruct((N, 3 * COMP), jnp.float32),
            jax.ShapeDtypeStruct((N, K), jnp.int32),
        ],
    )(cs, ce, multi, cxlo, cxhi, features, pos16, posT, bcol, brow,
      W_enc, b_enc, Wq, bq, Wk, bk, Wv, bv, Wp1p)


_NW = 32
_BPW = NK // _NW
_CH = 128
_NCH = _BPW // _CH
_TW = 3 * COMP


def _gather_body(kvp_hbm, idx_hbm, out_hbm,
                 idx_v, bufa, bufb, gsa, gsb, ssa, ssb):
    wid = lax.axis_index("s") * 2 + lax.axis_index("c")
    base = wid * _BPW
    pltpu.sync_copy(idx_hbm.at[pl.ds(base, _BPW)], idx_v)

    def pair(p, carry):
        ca = 2 * p
        cb = 2 * p + 1
        ga = pltpu.async_copy(
            kvp_hbm.at[idx_v.at[pl.ds(ca * _CH, _CH)]], bufa, gsa)
        gb = pltpu.async_copy(
            kvp_hbm.at[idx_v.at[pl.ds(cb * _CH, _CH)]], bufb, gsb)
        ga.wait()
        sa = pltpu.async_copy(bufa, out_hbm.at[pl.ds(base + ca * _CH, _CH)], ssa)
        gb.wait()
        sb = pltpu.async_copy(bufb, out_hbm.at[pl.ds(base + cb * _CH, _CH)], ssb)
        sa.wait()
        sb.wait()
        return carry

    lax.fori_loop(0, _NCH // 2, pair, 0)


def _gather(kvp, idx_flat):
    mesh = plsc.VectorSubcoreMesh(core_axis_name="c", subcore_axis_name="s")
    f = functools.partial(
        pl.kernel,
        out_type=jax.ShapeDtypeStruct((NK, _TW), jnp.float32),
        mesh=mesh,
        scratch_types=[
            pltpu.VMEM((_BPW,), jnp.int32),
            pltpu.VMEM((_CH, _TW), jnp.float32),
            pltpu.VMEM((_CH, _TW), jnp.float32),
            pltpu.SemaphoreType.DMA,
            pltpu.SemaphoreType.DMA,
            pltpu.SemaphoreType.DMA,
            pltpu.SemaphoreType.DMA,
        ],
    )(_gather_body)
    return f(kvp, idx_flat)


def _attn_body(kj_ref, vj_ref, pja_ref, q_ref, pia_ref, f_ref,
               bp1_ref, wp2_ref, bp2_ref,
               wa1_ref, ba1_ref, wa2_ref, ba2_ref,
               wd_ref, bd_ref, out_ref):
    nkb = PB * K
    piA = pia_ref[...]
    piAb = jnp.broadcast_to(piA[:, None, :], (PB, K, COMP)).reshape(nkb, COMP)
    pe_h = jnp.maximum(piAb - pja_ref[...] + bp1_ref[...], 0.0)
    pe = jnp.dot(pe_h, wp2_ref[...], preferred_element_type=jnp.float32) + bp2_ref[...]

    qb = jnp.broadcast_to(q_ref[...][:, None, :], (PB, K, COMP)).reshape(nkb, COMP)
    a = qb - kj_ref[...] + pe
    a_h = jnp.maximum(jnp.dot(a, wa1_ref[...], preferred_element_type=jnp.float32) + ba1_ref[...], 0.0)
    a = jnp.dot(a_h, wa2_ref[...], preferred_element_type=jnp.float32) + ba2_ref[...]

    a3 = a.reshape(PB, K, COMP)
    mx = jnp.max(a3, axis=1, keepdims=True)
    e = jnp.exp(a3 - mx)
    s = jnp.sum(e, axis=1, keepdims=True)
    w = e / s
    vpe = (vj_ref[...] + pe).reshape(PB, K, COMP)
    agg = jnp.sum(w * vpe, axis=1)

    y = jnp.dot(agg, wd_ref[...], preferred_element_type=jnp.float32) + bd_ref[...]
    out_ref[...] = f_ref[...] + y


def _attention(kvpj, q, kvp, features,
               bp1, Wp2, bp2, Wa1, ba1, Wa2, ba2, W_dec, b_dec):
    grid = (N // PB,)
    full = lambda shape: pl.BlockSpec(shape, lambda i: (0, 0))
    nkcol = lambda c: pl.BlockSpec((PB * K, COMP), lambda i, c=c: (i, c))
    pblk = lambda w: pl.BlockSpec((PB, w), lambda i: (i, 0))
    return pl.pallas_call(
        _attn_body,
        grid=grid,
        in_specs=[
            nkcol(0), nkcol(1), nkcol(2),
            pblk(COMP),
            pl.BlockSpec((PB, COMP), lambda i: (i, 2)),
            pblk(IN_F),
            full((1, COMP)),
            full((COMP, COMP)), full((1, COMP)),
            full((COMP, COMP)), full((1, COMP)),
            full((COMP, COMP)), full((1, COMP)),
            full((COMP, IN_F)), full((1, IN_F)),
        ],
        out_specs=pblk(IN_F),
        out_shape=jax.ShapeDtypeStruct((N, IN_F), jnp.float32),
    )(kvpj, kvpj, kvpj, q, kvp, features,
      bp1, Wp2, bp2, Wa1, ba1, Wa2, ba2, W_dec, b_dec)


def kernel(features, positions, batch, W_enc, b_enc, Wq, bq, Wk, bk, Wv, bv,
           Wp1, bp1, Wp2, bp2, Wa1, ba1, Wa2, ba2, W_dec, b_dec):
    _, _, perm = lax.sort(
        (batch, positions[:, 0], jnp.arange(N, dtype=jnp.int32)), num_keys=2)
    sf = jnp.take(features, perm, axis=0)
    sp = jnp.take(positions, perm, axis=0)
    sb = jnp.take(batch, perm)

    pos16 = jnp.pad(sp, ((0, 0), (0, PPAD - 3)))
    posT = pos16.T
    batchf = sb.astype(jnp.float32)
    bcol = batchf.reshape(N, 1)
    brow = batchf.reshape(1, N)
    row = lambda b: b.reshape(1, -1)

    Wp1p = jnp.pad(Wp1, ((0, PPAD - 3), (0, 0)))
    blk = jnp.arange(N // RB)
    b_lo = sb[blk * RB]
    b_hi = sb[blk * RB + RB - 1]
    col_start = jnp.searchsorted(sb, b_lo, side="left")
    col_end = jnp.searchsorted(sb, b_hi, side="right")
    cs = (col_start // CW).astype(jnp.int32)
    ce = ((col_end + CW - 1) // CW).astype(jnp.int32)
    multi = (b_lo != b_hi).astype(jnp.int32)
    xs = sp[:, 0].reshape(NCH, CW)
    cxlo = jnp.full((CPAD,), jnp.inf, jnp.float32).at[:NCH].set(xs.min(axis=1))
    cxhi = jnp.full((CPAD,), -jnp.inf, jnp.float32).at[:NCH].set(xs.max(axis=1))
    q, kvp, idx = _encode_knn(
        cs, ce, multi, cxlo, cxhi, sf, pos16, posT, bcol, brow,
        W_enc, row(b_enc), Wq, row(bq), Wk, row(bk), Wv, row(bv), Wp1p)

    idx_flat = idx.reshape(NK)
    kvpj = _gather(kvp, idx_flat)

    out_s = _attention(kvpj, q, kvp, sf,
                       row(bp1), Wp2, row(bp2),
                       Wa1, row(ba1), Wa2, row(ba2), W_dec, row(b_dec))
    out = jnp.zeros_like(out_s).at[perm].set(out_s)
    return (out, positions, batch)


# speedup vs baseline: 1.6088x; 1.6088x over previous
import functools

import jax
import jax.numpy as jnp
from jax import lax
from jax.experimental import pallas as pl
from jax.experimental.pallas import tpu as pltpu
from jax.experimental.pallas import tpu_sc as plsc

N = 8192
IN_F = 512
COMP = 128
K = 16
PPAD = 16
RB = 256
PB = 256
NK = N * K


CW = 512


def _encode_knn_body(cs_ref, ce_ref,
                     f_ref, posb_ref, posT_ref, bcol_ref, brow_ref,
                     we_ref, be_ref, wq_ref, bq_ref, wk_ref, bk_ref,
                     wv_ref, bv_ref, wp1_ref,
                     q_ref, kvp_ref, idx_ref):
    h = jnp.dot(f_ref[...], we_ref[...], preferred_element_type=jnp.float32)
    h = h + be_ref[...]
    q_ref[...] = jnp.dot(h, wq_ref[...], preferred_element_type=jnp.float32) + bq_ref[...]
    kvp_ref[:, 0:COMP] = jnp.dot(h, wk_ref[...], preferred_element_type=jnp.float32) + bk_ref[...]
    kvp_ref[:, COMP:2 * COMP] = jnp.dot(h, wv_ref[...], preferred_element_type=jnp.float32) + bv_ref[...]

    posb = posb_ref[...]
    kvp_ref[:, 2 * COMP:3 * COMP] = jnp.dot(posb, wp1_ref[...], preferred_element_type=jnp.float32)
    sqb = jnp.sum(posb * posb, axis=1, keepdims=True)
    bcol = bcol_ref[...]
    i = pl.program_id(0)
    cs = cs_ref[i]
    ce = ce_ref[i]

    init_v = jnp.full((RB, K), jnp.inf, jnp.float32)
    init_i = jnp.broadcast_to(
        (-1.0 - lax.broadcasted_iota(jnp.int32, (1, K), 1).astype(jnp.float32)),
        (RB, K))
    iota_c = lax.broadcasted_iota(jnp.int32, (RB, CW), 1)

    def chunk_body(c, carry):
        vals, idxs = carry
        pcols = posT_ref[:, pl.ds(c * CW, CW)]
        dots = jnp.dot(posb, pcols, preferred_element_type=jnp.float32)
        sqr = jnp.sum(pcols * pcols, axis=0, keepdims=True)
        d2 = sqb + sqr - 2.0 * dots
        same = bcol == brow_ref[:, pl.ds(c * CW, CW)]
        cand = jnp.where(same, d2, jnp.inf)
        cidx = (c * CW + iota_c).astype(jnp.float32)
        wv = jnp.concatenate([vals, cand], axis=1)
        wi = jnp.concatenate([idxs, cidx], axis=1)
        nv, ni = [], []
        for _ in range(K):
            m = jnp.min(wv, axis=1, keepdims=True)
            tie = jnp.where(wv == m, wi, jnp.float32(N))
            amin = jnp.min(tie, axis=1, keepdims=True)
            nv.append(m)
            ni.append(amin)
            wv = jnp.where(tie == amin, jnp.inf, wv)
        return jnp.concatenate(nv, axis=1), jnp.concatenate(ni, axis=1)

    vals, idxs = lax.fori_loop(cs, ce, chunk_body, (init_v, init_i))
    idx_ref[...] = jnp.clip(idxs, 0.0, jnp.float32(N - 1)).astype(jnp.int32)


def _encode_knn(cs, ce, features, pos16, posT, bcol, brow, W_enc, b_enc,
                Wq, bq, Wk, bk, Wv, bv, Wp1p):
    grid = (N // RB,)
    full = lambda shape: pl.BlockSpec(shape, lambda i, s0, s1: (0, 0))
    rowblk = lambda w: pl.BlockSpec((RB, w), lambda i, s0, s1: (i, 0))
    return pl.pallas_call(
        _encode_knn_body,
        grid_spec=pltpu.PrefetchScalarGridSpec(
            num_scalar_prefetch=2,
            grid=grid,
            in_specs=[
                rowblk(IN_F),
                rowblk(PPAD),
                full((PPAD, N)),
                pl.BlockSpec((RB, 1), lambda i, s0, s1: (i, 0)),
                full((1, N)),
                full((IN_F, COMP)), full((1, COMP)),
                full((COMP, COMP)), full((1, COMP)),
                full((COMP, COMP)), full((1, COMP)),
                full((COMP, COMP)), full((1, COMP)),
                full((PPAD, COMP)),
            ],
            out_specs=[
                rowblk(COMP), rowblk(3 * COMP),
                pl.BlockSpec((RB, K), lambda i, s0, s1: (i, 0)),
            ],
        ),
        out_shape=[
            jax.ShapeDtypeStruct((N, COMP), jnp.float32),
            jax.ShapeDtypeStruct((N, 3 * COMP), jnp.float32),
            jax.ShapeDtypeStruct((N, K), jnp.int32),
        ],
    )(cs, ce, features, pos16, posT, bcol, brow, W_enc, b_enc,
      Wq, bq, Wk, bk, Wv, bv, Wp1p)


_NW = 32
_BPW = NK // _NW
_CH = 128
_NCH = _BPW // _CH
_TW = 3 * COMP


def _gather_body(kvp_hbm, idx_hbm, out_hbm,
                 idx_v, bufa, bufb, gsa, gsb, ssa, ssb):
    wid = lax.axis_index("s") * 2 + lax.axis_index("c")
    base = wid * _BPW
    pltpu.sync_copy(idx_hbm.at[pl.ds(base, _BPW)], idx_v)

    def pair(p, carry):
        ca = 2 * p
        cb = 2 * p + 1
        ga = pltpu.async_copy(
            kvp_hbm.at[idx_v.at[pl.ds(ca * _CH, _CH)]], bufa, gsa)
        gb = pltpu.async_copy(
            kvp_hbm.at[idx_v.at[pl.ds(cb * _CH, _CH)]], bufb, gsb)
        ga.wait()
        sa = pltpu.async_copy(bufa, out_hbm.at[pl.ds(base + ca * _CH, _CH)], ssa)
        gb.wait()
        sb = pltpu.async_copy(bufb, out_hbm.at[pl.ds(base + cb * _CH, _CH)], ssb)
        sa.wait()
        sb.wait()
        return carry

    lax.fori_loop(0, _NCH // 2, pair, 0)


def _gather(kvp, idx_flat):
    mesh = plsc.VectorSubcoreMesh(core_axis_name="c", subcore_axis_name="s")
    f = functools.partial(
        pl.kernel,
        out_type=jax.ShapeDtypeStruct((NK, _TW), jnp.float32),
        mesh=mesh,
        scratch_types=[
            pltpu.VMEM((_BPW,), jnp.int32),
            pltpu.VMEM((_CH, _TW), jnp.float32),
            pltpu.VMEM((_CH, _TW), jnp.float32),
            pltpu.SemaphoreType.DMA,
            pltpu.SemaphoreType.DMA,
            pltpu.SemaphoreType.DMA,
            pltpu.SemaphoreType.DMA,
        ],
    )(_gather_body)
    return f(kvp, idx_flat)


def _attn_body(kj_ref, vj_ref, pja_ref, q_ref, pia_ref, f_ref,
               bp1_ref, wp2_ref, bp2_ref,
               wa1_ref, ba1_ref, wa2_ref, ba2_ref,
               wd_ref, bd_ref, out_ref):
    nkb = PB * K
    piA = pia_ref[...]
    piAb = jnp.broadcast_to(piA[:, None, :], (PB, K, COMP)).reshape(nkb, COMP)
    pe_h = jnp.maximum(piAb - pja_ref[...] + bp1_ref[...], 0.0)
    pe = jnp.dot(pe_h, wp2_ref[...], preferred_element_type=jnp.float32) + bp2_ref[...]

    qb = jnp.broadcast_to(q_ref[...][:, None, :], (PB, K, COMP)).reshape(nkb, COMP)
    a = qb - kj_ref[...] + pe
    a_h = jnp.maximum(jnp.dot(a, wa1_ref[...], preferred_element_type=jnp.float32) + ba1_ref[...], 0.0)
    a = jnp.dot(a_h, wa2_ref[...], preferred_element_type=jnp.float32) + ba2_ref[...]

    a3 = a.reshape(PB, K, COMP)
    mx = jnp.max(a3, axis=1, keepdims=True)
    e = jnp.exp(a3 - mx)
    s = jnp.sum(e, axis=1, keepdims=True)
    w = e / s
    vpe = (vj_ref[...] + pe).reshape(PB, K, COMP)
    agg = jnp.sum(w * vpe, axis=1)

    y = jnp.dot(agg, wd_ref[...], preferred_element_type=jnp.float32) + bd_ref[...]
    out_ref[...] = f_ref[...] + y


def _attention(kvpj, q, kvp, features,
               bp1, Wp2, bp2, Wa1, ba1, Wa2, ba2, W_dec, b_dec):
    grid = (N // PB,)
    full = lambda shape: pl.BlockSpec(shape, lambda i: (0, 0))
    nkcol = lambda c: pl.BlockSpec((PB * K, COMP), lambda i, c=c: (i, c))
    pblk = lambda w: pl.BlockSpec((PB, w), lambda i: (i, 0))
    return pl.pallas_call(
        _attn_body,
        grid=grid,
        in_specs=[
            nkcol(0), nkcol(1), nkcol(2),
            pblk(COMP),
            pl.BlockSpec((PB, COMP), lambda i: (i, 2)),
            pblk(IN_F),
            full((1, COMP)),
            full((COMP, COMP)), full((1, COMP)),
            full((COMP, COMP)), full((1, COMP)),
            full((COMP, COMP)), full((1, COMP)),
            full((COMP, IN_F)), full((1, IN_F)),
        ],
        out_specs=pblk(IN_F),
        out_shape=jax.ShapeDtypeStruct((N, IN_F), jnp.float32),
    )(kvpj, kvpj, kvpj, q, kvp, features,
      bp1, Wp2, bp2, Wa1, ba1, Wa2, ba2, W_dec, b_dec)


def kernel(features, positions, batch, W_enc, b_enc, Wq, bq, Wk, bk, Wv, bv,
           Wp1, bp1, Wp2, bp2, Wa1, ba1, Wa2, ba2, W_dec, b_dec):
    pos16 = jnp.pad(positions, ((0, 0), (0, PPAD - 3)))
    posT = pos16.T
    batchf = batch.astype(jnp.float32)
    bcol = batchf.reshape(N, 1)
    brow = batchf.reshape(1, N)
    row = lambda b: b.reshape(1, -1)

    Wp1p = jnp.pad(Wp1, ((0, PPAD - 3), (0, 0)))
    blk = jnp.arange(N // RB)
    b_lo = batch[blk * RB]
    b_hi = batch[blk * RB + RB - 1]
    col_start = jnp.searchsorted(batch, b_lo, side="left")
    col_end = jnp.searchsorted(batch, b_hi, side="right")
    cs = (col_start // CW).astype(jnp.int32)
    ce = ((col_end + CW - 1) // CW).astype(jnp.int32)
    q, kvp, idx = _encode_knn(
        cs, ce, features, pos16, posT, bcol, brow, W_enc, row(b_enc),
        Wq, row(bq), Wk, row(bk), Wv, row(bv), Wp1p)

    idx_flat = idx.reshape(NK)
    kvpj = _gather(kvp, idx_flat)

    out = _attention(kvpj, q, kvp, features,
                     row(bp1), Wp2, row(bp2),
                     Wa1, row(ba1), Wa2, row(ba2), W_dec, row(b_dec))
    return (out, positions, batch)
